# baseline (speedup 1.0000x reference)
import functools

import jax
import jax.numpy as jnp
from jax import lax
from jax.experimental import pallas as pl
from jax.experimental.pallas import tpu as pltpu
from jax.experimental.pallas import tpu_sc as plsc

_NC = 2
_NS = 16
_NW = _NC * _NS
_CHUNK = 128
_SKEW = 1


@functools.lru_cache(maxsize=None)
def _make_sc_gather_t(V, D, B):
    bpw = B // _NW
    n_chunks = bpw // _CHUNK
    pstride = bpw + _SKEW
    mesh = plsc.VectorSubcoreMesh(core_axis_name="c", subcore_axis_name="s")

    @functools.partial(
        pl.kernel,
        mesh=mesh,
        out_type=jax.ShapeDtypeStruct((D, B), jnp.float32),
        scratch_types=[
            pltpu.VMEM((bpw,), jnp.int32),
            pltpu.VMEM((bpw, D), jnp.float32),
            pltpu.VMEM((D, pstride), jnp.float32),
            pltpu.SemaphoreType.DMA,
            pltpu.SemaphoreType.DMA,
        ],
        compiler_params=pltpu.CompilerParams(
            use_tc_tiling_on_sc=False, needs_layout_passes=False
        ),
    )
    def sc_gather_t(table_hbm, idx_hbm, out_hbm, idx_v, rows_v, panel, gsem, wsem):
        wid = lax.axis_index("s") * _NC + lax.axis_index("c")
        base = wid * bpw
        pltpu.sync_copy(idx_hbm.at[pl.ds(base, bpw)], idx_v)
        gathers = []
        for c in range(n_chunks):
            gathers.append(
                pltpu.async_copy(
                    table_hbm.at[idx_v.at[pl.ds(c * _CHUNK, _CHUNK)]],
                    rows_v.at[pl.ds(c * _CHUNK, _CHUNK)],
                    gsem,
                )
            )
        lane = lax.iota(jnp.int32, 16)
        col_ids = [lane + cg * 16 for cg in range(D // 16)]
        writes = []
        for c in range(n_chunks):
            gathers[c].wait()

            def jg_body(jg, _):
                j0 = c * _CHUNK + jg * 16
                for r in range(16):
                    j = j0 + r
                    j_vec = jnp.full((16,), j, jnp.int32)
                    for cg in range(D // 16):
                        v = rows_v[j, pl.ds(cg * 16, 16)]
                        plsc.store_scatter(panel, [col_ids[cg], j_vec], v)
                return 0

            lax.fori_loop(0, _CHUNK // 16, jg_body, 0)
            writes.append(
                pltpu.async_copy(
                    panel.at[:, pl.ds(c * _CHUNK, _CHUNK)],
                    out_hbm.at[:, pl.ds(base + c * _CHUNK, _CHUNK)],
                    wsem,
                )
            )
        for w in writes:
            w.wait()

    return sc_gather_t


def kernel(speaker, embedding_table):
    idx = speaker.astype(jnp.int32)
    (B,) = idx.shape
    V, D = embedding_table.shape
    out_t = _make_sc_gather_t(V, D, B)(embedding_table, idx)
    return out_t.T


# speedup vs baseline: 1.0136x; 1.0136x over previous
import functools

import jax
import jax.numpy as jnp
from jax import lax
from jax.experimental import pallas as pl
from jax.experimental.pallas import tpu as pltpu
from jax.experimental.pallas import tpu_sc as plsc

_NC = 2
_NS = 16
_NW = _NC * _NS
_CHUNK = 128


@functools.lru_cache(maxsize=None)
def _make_sc_gather_tiled(V, D, B):
    bpw = B // _NW
    n_chunks = bpw // _CHUNK
    n_tr = D // 8
    n_tc = B // 128
    w_tc = bpw // 128
    mesh = plsc.VectorSubcoreMesh(core_axis_name="c", subcore_axis_name="s")

    @functools.partial(
        pl.kernel,
        mesh=mesh,
        out_type=jax.ShapeDtypeStruct((n_tr, n_tc, 1024), jnp.float32),
        scratch_types=[
            pltpu.VMEM((bpw,), jnp.int32),
            pltpu.VMEM((bpw, D), jnp.float32),
            pltpu.VMEM((n_tr * w_tc * 1024,), jnp.float32),
            pltpu.SemaphoreType.DMA,
            pltpu.SemaphoreType.DMA,
        ],
        compiler_params=pltpu.CompilerParams(
            use_tc_tiling_on_sc=False, needs_layout_passes=False
        ),
    )
    def sc_gather_t(table_hbm, idx_hbm, out_hbm, idx_v, rows_v, panel, gsem, wsem):
        wid = lax.axis_index("s") * _NC + lax.axis_index("c")
        base = wid * bpw
        pltpu.sync_copy(idx_hbm.at[pl.ds(base, bpw)], idx_v)
        gathers = []
        for c in range(n_chunks):
            gathers.append(
                pltpu.async_copy(
                    table_hbm.at[idx_v.at[pl.ds(c * _CHUNK, _CHUNK)]],
                    rows_v.at[pl.ds(c * _CHUNK, _CHUNK)],
                    gsem,
                )
            )
        lane = lax.iota(jnp.int32, 16)
        diag = [(lane + i) % 16 for i in range(16)]
        epart = [(dg // 8) * (w_tc * 1024) + (dg % 8) * 128 for dg in diag]
        writes = []
        for c in range(n_chunks):
            gathers[c].wait()

            def jg_body(jg, _):
                j_loc = c * _CHUNK + jg * 16 + lane
                col_part = c * 1024 + jg * 16 + lane
                for cg in range(D // 16):
                    gbase = col_part + cg * (2 * w_tc * 1024)
                    for i in range(16):
                        d_vec = diag[i] + cg * 16
                        v = plsc.load_gather(rows_v, [j_loc, d_vec])
                        plsc.store_scatter(panel, [gbase + epart[i]], v)
                return 0

            lax.fori_loop(0, _CHUNK // 16, jg_body, 0)
            for tr in range(n_tr):
                writes.append(
                    pltpu.async_copy(
                        panel.at[pl.ds(tr * (w_tc * 1024) + c * 1024, 1024)],
                        out_hbm.at[tr, wid * w_tc + c],
                        wsem,
                    )
                )
        for w in writes:
            w.wait()

    return sc_gather_t


def kernel(speaker, embedding_table):
    idx = speaker.astype(jnp.int32)
    (B,) = idx.shape
    V, D = embedding_table.shape
    x = _make_sc_gather_tiled(V, D, B)(embedding_table, idx)
    out_t = x.reshape(D // 8, B // 128, 8, 128).transpose(0, 2, 1, 3).reshape(D, B)
    return out_t.T


# speedup vs baseline: 1.1645x; 1.1488x over previous
import functools

import jax
import jax.numpy as jnp
from jax import lax
from jax.experimental import pallas as pl
from jax.experimental.pallas import tpu as pltpu
from jax.experimental.pallas import tpu_sc as plsc

_NC = 2
_NS = 16
_NW = _NC * _NS
_CHUNK = 128


@functools.lru_cache(maxsize=None)
def _make_sc_gather(V, D, B):
    bpw = B // _NW
    n_chunks = bpw // _CHUNK
    mesh = plsc.VectorSubcoreMesh(core_axis_name="c", subcore_axis_name="s")

    @functools.partial(
        pl.kernel,
        mesh=mesh,
        out_type=jax.ShapeDtypeStruct((5 * B, 2 * D), jnp.float32),
        scratch_types=[
            pltpu.VMEM((bpw,), jnp.int32),
            pltpu.VMEM((bpw, D), jnp.float32),
            pltpu.SemaphoreType.DMA,
        ],
        compiler_params=pltpu.CompilerParams(use_tc_tiling_on_sc=False),
    )
    def sc_gather(table_hbm, idx_hbm, out_hbm, idx_v, rows_v, sem):
        wid = lax.axis_index("s") * _NC + lax.axis_index("c")
        base = wid * bpw
        pltpu.sync_copy(idx_hbm.at[pl.ds(base, bpw)], idx_v)
        copies = []
        for c in range(n_chunks):
            copies.append(
                pltpu.async_copy(
                    table_hbm.at[idx_v.at[pl.ds(c * _CHUNK, _CHUNK)]],
                    rows_v.at[pl.ds(c * _CHUNK, _CHUNK)],
                    sem,
                )
            )
        for cp in copies:
            cp.wait()
        pltpu.sync_copy(rows_v, out_hbm.at[pl.ds(base, bpw), pl.ds(0, D)])

    return sc_gather


@functools.lru_cache(maxsize=None)
def _make_transpose(D, B, blk=2048):
    n = B // blk

    def body(in_hbm, out_ref, buf, sems):
        i = pl.program_id(0)
        slot = lax.rem(i, 2)
        nxt = lax.rem(i + 1, 2)

        @pl.when(i == 0)
        def _():
            pltpu.make_async_copy(
                in_hbm.at[pl.ds(0, blk), pl.ds(0, 2 * D)], buf.at[0], sems.at[0]
            ).start()

        @pl.when(i < n - 1)
        def _():
            pltpu.make_async_copy(
                in_hbm.at[pl.ds((i + 1) * blk, blk), pl.ds(0, 2 * D)],
                buf.at[nxt],
                sems.at[nxt],
            ).start()

        pltpu.make_async_copy(
            in_hbm.at[pl.ds(i * blk, blk), pl.ds(0, 2 * D)], buf.at[slot], sems.at[slot]
        ).wait()
        out_ref[...] = buf[slot].T[:D, :]

    def run(x):
        return pl.pallas_call(
            body,
            out_shape=jax.ShapeDtypeStruct((D, B), jnp.float32),
            grid=(n,),
            in_specs=[pl.BlockSpec(memory_space=pl.ANY)],
            out_specs=pl.BlockSpec((D, blk), lambda i: (0, i)),
            scratch_shapes=[
                pltpu.VMEM((2, blk, 2 * D), jnp.float32),
                pltpu.SemaphoreType.DMA((2,)),
            ],
        )(x)

    return run


def kernel(speaker, embedding_table):
    idx = speaker.astype(jnp.int32)
    (B,) = idx.shape
    V, D = embedding_table.shape
    staged = _make_sc_gather(V, D, B)(embedding_table, idx)
    out_t = _make_transpose(D, B)(staged)
    return out_t.T


# speedup vs baseline: 1.2783x; 1.0977x over previous
import functools

import jax
import jax.numpy as jnp
from jax import lax
from jax.experimental import pallas as pl
from jax.experimental.pallas import tpu as pltpu
from jax.experimental.pallas import tpu_sc as plsc

_NC = 2
_NS = 16
_NW = _NC * _NS
_CHUNK = 128


@functools.lru_cache(maxsize=None)
def _make_sc_gather(V, D, B):
    bpw = B // _NW
    n_chunks = bpw // _CHUNK
    mesh = plsc.VectorSubcoreMesh(core_axis_name="c", subcore_axis_name="s")

    @functools.partial(
        pl.kernel,
        mesh=mesh,
        out_type=jax.ShapeDtypeStruct((5 * B, 2 * D), jnp.float32),
        scratch_types=[
            pltpu.VMEM((bpw,), jnp.int32),
            pltpu.VMEM((bpw, D), jnp.float32),
            pltpu.SemaphoreType.DMA,
        ],
        compiler_params=pltpu.CompilerParams(use_tc_tiling_on_sc=False),
    )
    def sc_gather(table_hbm, idx_hbm, out_hbm, idx_v, rows_v, sem):
        wid = lax.axis_index("s") * _NC + lax.axis_index("c")
        base = wid * bpw
        pltpu.sync_copy(idx_hbm.at[pl.ds(base, bpw)], idx_v)
        copies = []
        for c in range(n_chunks):
            copies.append(
                pltpu.async_copy(
                    table_hbm.at[idx_v.at[pl.ds(c * _CHUNK, _CHUNK)]],
                    rows_v.at[pl.ds(c * _CHUNK, _CHUNK)],
                    sem,
                )
            )
        for cp in copies:
            cp.wait()
        pltpu.sync_copy(rows_v, out_hbm.at[pl.ds(base, bpw), pl.ds(0, D)])

    return sc_gather


@functools.lru_cache(maxsize=None)
def _make_transpose(D, B, blk=2048):
    n = B // blk

    def body(in_hbm, out_ref, buf, sems):
        i = pl.program_id(0)

        @pl.when(i == 0)
        def _():
            for k in range(n):
                pltpu.make_async_copy(
                    in_hbm.at[pl.ds(k * blk, blk), pl.ds(0, 2 * D)],
                    buf.at[k],
                    sems.at[k],
                ).start()

        pltpu.make_async_copy(
            in_hbm.at[pl.ds(i * blk, blk), pl.ds(0, 2 * D)], buf.at[i], sems.at[i]
        ).wait()
        out_ref[...] = buf[i].T[:D, :]

    def run(x):
        return pl.pallas_call(
            body,
            out_shape=jax.ShapeDtypeStruct((D, B), jnp.float32),
            grid=(n,),
            in_specs=[pl.BlockSpec(memory_space=pl.ANY)],
            out_specs=pl.BlockSpec((D, blk), lambda i: (0, i)),
            scratch_shapes=[
                pltpu.VMEM((n, blk, 2 * D), jnp.float32),
                pltpu.SemaphoreType.DMA((n,)),
            ],
        )(x)

    return run


def kernel(speaker, embedding_table):
    idx = speaker.astype(jnp.int32)
    (B,) = idx.shape
    V, D = embedding_table.shape
    staged = _make_sc_gather(V, D, B)(embedding_table, idx)
    out_t = _make_transpose(D, B)(staged)
    return out_t.T
